# baseline (speedup 1.0000x reference)
import jax
import jax.numpy as jnp
from jax import lax
from jax.experimental import pallas as pl
from jax.experimental.pallas import tpu as pltpu
from jax.experimental.pallas import tpu_sc as plsc

N = 10000
E = 320000
D = 128
H = 512

NC = 2
NS = 16
NW = NC * NS
NP = 10240

EPT = 10240
EP = NW * EPT
K = 64
G = EPT // K
RPT = NP // NS
BR = 640
GRID = NP // BR

f32 = jnp.float32


def _mesh():
    return plsc.VectorSubcoreMesh(core_axis_name="c", subcore_axis_name="s")


def _bincount_body(src_hbm, dst_hbm, out_hbm, eidx_s, eidx_d, acc_s, acc_d):
    cid = lax.axis_index("c")
    sid = lax.axis_index("s")
    w = cid * NS + sid
    pltpu.sync_copy(src_hbm.at[pl.ds(w * EPT, EPT)], eidx_s)
    pltpu.sync_copy(dst_hbm.at[pl.ds(w * EPT, EPT)], eidx_d)

    zeros16 = jnp.zeros((16,), f32)

    def zero_blk(i, c):
        acc_s[pl.ds(i * 16, 16)] = zeros16
        acc_d[pl.ds(i * 16, 16)] = zeros16
        return c
    lax.fori_loop(0, NP // 16, zero_blk, 0, unroll=False)

    ones16 = jnp.ones((16,), f32)

    def count_blk(i, c):
        plsc.addupdate_scatter(acc_s, [eidx_s[pl.ds(i * 16, 16)]], ones16)
        plsc.addupdate_scatter(acc_d, [eidx_d[pl.ds(i * 16, 16)]], ones16)
        return c
    lax.fori_loop(0, EPT // 16, count_blk, 0, unroll=False)

    pltpu.sync_copy(acc_s, out_hbm.at[pl.ds(w * NP, NP)])
    pltpu.sync_copy(acc_d, out_hbm.at[pl.ds((NW + w) * NP, NP)])


_bincount = pl.kernel(
    _bincount_body,
    out_type=jax.ShapeDtypeStruct((2 * NW * NP,), f32),
    mesh=_mesh(),
    compiler_params=pltpu.CompilerParams(needs_layout_passes=False),
    scratch_types=[
        pltpu.VMEM((EPT,), jnp.int32),
        pltpu.VMEM((EPT,), jnp.int32),
        pltpu.VMEM((NP,), f32),
        pltpu.VMEM((NP,), f32),
    ],
)


def _make_agg(T):
    def body(src_hbm, dst_hbm, zeros_hbm, *rest):
        tables = rest[:T]
        outs = rest[T:2 * T]
        idx_s, idx_d, rows, sem0, sem1, acc = rest[2 * T:]
        cid = lax.axis_index("c")
        sid = lax.axis_index("s")
        w = cid * NS + sid
        pltpu.sync_copy(src_hbm.at[pl.ds(w * EPT, EPT)], idx_s)
        pltpu.sync_copy(dst_hbm.at[pl.ds(w * G, G)], idx_d)

        def src_at(g):
            return idx_s.at[pl.ds(g * K, K)]

        for t in range(T):
            pltpu.sync_copy(zeros_hbm, acc.at[pl.ds(sid * RPT, RPT)])
            plsc.subcore_barrier()
            tbl = tables[t]
            pltpu.async_copy(tbl.at[src_at(0)], rows.at[0], sem0)

            def pair(h, c):
                g = 2 * h
                pltpu.async_copy(tbl.at[src_at(g + 1)], rows.at[1], sem1)
                pltpu.make_async_copy(tbl.at[src_at(g)], rows.at[0],
                                      sem0).wait()
                pltpu.sync_copy(rows.at[0], acc.at[idx_d.at[g]], add=True)
                gn = jnp.minimum(g + 2, G - 2)
                pltpu.async_copy(tbl.at[src_at(gn)], rows.at[0], sem0)
                pltpu.make_async_copy(tbl.at[src_at(g + 1)], rows.at[1],
                                      sem1).wait()
                pltpu.sync_copy(rows.at[1], acc.at[idx_d.at[g + 1]], add=True)
                return c
            lax.fori_loop(0, G // 2, pair, 0, unroll=False)
            pltpu.make_async_copy(tbl.at[src_at(0)], rows.at[0],
                                  sem0).wait()
            plsc.subcore_barrier()
            pltpu.sync_copy(acc.at[pl.ds(sid * RPT, RPT)],
                            outs[t].at[pl.ds(cid * NP + sid * RPT, RPT)])
    out_type = [jax.ShapeDtypeStruct((2 * NP, D), f32)] * T
    if T == 1:
        out_type = out_type[0]
    return pl.kernel(
        body,
        out_type=out_type,
        mesh=_mesh(),
        compiler_params=pltpu.CompilerParams(needs_layout_passes=False),
        scratch_types=[
            pltpu.VMEM((EPT,), jnp.int32),
            pltpu.VMEM((G, K), jnp.int32),
            pltpu.VMEM((2, K, D), f32),
            pltpu.SemaphoreType.DMA,
            pltpu.SemaphoreType.DMA,
            pltpu.VMEM_SHARED((NP, D), f32),
        ],
    )


_agg1 = _make_agg(1)
_agg4 = _make_agg(4)


def _tc_scale_body(dego, degi, x, xs, so, si):
    so_v = lax.rsqrt(jnp.maximum(
        jnp.sum(dego[...], axis=1, keepdims=True), 1.0))
    si_v = lax.rsqrt(jnp.maximum(
        jnp.sum(degi[...], axis=1, keepdims=True), 1.0))
    xs[...] = x[...] * so_v
    so[...] = so_v
    si[...] = si_v


def _row_spec(width):
    return pl.BlockSpec((BR, width), lambda i: (i, 0))


def _row_spec_off(width, off):
    return pl.BlockSpec((BR, width), lambda i, o=off: (i + o, 0))


def _full_spec(r, c):
    return pl.BlockSpec((r, c), lambda i: (0, 0))


_tc_scale = pl.pallas_call(
    _tc_scale_body,
    grid=(GRID,),
    in_specs=[_row_spec(NW), _row_spec(NW), _row_spec(D)],
    out_specs=[_row_spec(D), _row_spec(1), _row_spec(1)],
    out_shape=[jax.ShapeDtypeStruct((NP, D), f32),
               jax.ShapeDtypeStruct((NP, 1), f32),
               jax.ShapeDtypeStruct((NP, 1), f32)],
)


def _tc_l1_body(p0, p1, si, so, W1, b1, a1, out):
    agg = (p0[...] + p1[...]) * si[...]
    z = jnp.dot(agg, W1[...], preferred_element_type=f32) + b1[...]
    a = a1[0, 0]
    h = jnp.where(z > 0, z, a * z)
    hs = h * so[...]
    for c in range(4):
        out[c] = hs[:, c * 128:(c + 1) * 128]


_tc_l1 = pl.pallas_call(
    _tc_l1_body,
    grid=(GRID,),
    in_specs=[_row_spec(D), _row_spec_off(D, GRID), _row_spec(1), _row_spec(1),
              _full_spec(D, H), _full_spec(1, H), _full_spec(1, 1)],
    out_specs=pl.BlockSpec((4, BR, 128), lambda i: (0, i, 0)),
    out_shape=jax.ShapeDtypeStruct((4, NP, 128), f32),
)


def _tc_wc_body(We2d, Wd, Wc):
    Wc[...] = jnp.dot(We2d[...], Wd[...], preferred_element_type=f32)


_tc_wc = pl.pallas_call(
    _tc_wc_body,
    out_shape=jax.ShapeDtypeStruct((H, D), f32),
)


def _tc_l2_body(q00, q01, q10, q11, q20, q21, q30, q31, si, so, W2, b2, a2,
                Wc, out):
    siv = si[...]
    W2v = W2[...]
    qs = ((q00, q01), (q10, q11), (q20, q21), (q30, q31))
    z = jnp.zeros((BR, H), f32) + b2[...]
    for c in range(4):
        agg_c = (qs[c][0][...] + qs[c][1][...]) * siv
        z = z + jnp.dot(agg_c, W2v[c * 128:(c + 1) * 128, :],
                        preferred_element_type=f32)
    a = a2[0, 0]
    h = jnp.where(z > 0, z, a * z)
    t = jnp.dot(h, Wc[...], preferred_element_type=f32)
    out[...] = t * so[...]


_tc_l2 = pl.pallas_call(
    _tc_l2_body,
    grid=(GRID,),
    in_specs=[_row_spec(D), _row_spec_off(D, GRID)] * 4
    + [_row_spec(1), _row_spec(1), _full_spec(H, H), _full_spec(1, H),
       _full_spec(1, 1), _full_spec(H, D)],
    out_specs=_row_spec(D),
    out_shape=jax.ShapeDtypeStruct((NP, D), f32),
)


def _tc_fin_body(r0, r1, si, bd, out):
    out[...] = (r0[...] + r1[...]) * si[...] + bd[...]


_tc_fin = pl.pallas_call(
    _tc_fin_body,
    grid=(GRID,),
    in_specs=[_row_spec(D), _row_spec_off(D, GRID), _row_spec(1),
              _full_spec(1, D)],
    out_specs=_row_spec(D),
    out_shape=jax.ShapeDtypeStruct((NP, D), f32),
)


def kernel(x, edge_index, W1, b1, a1, W2, b2, a2, We2d, Wd, bd):
    src = edge_index[0]
    dst = edge_index[1]
    pad = jnp.full((EP - E,), NP - 1, jnp.int32)
    src_f = jnp.concatenate([src, pad])
    dst_f = jnp.concatenate([dst, pad])
    dst_a = dst_f.reshape(NW * G, K)
    zeros = jnp.zeros((RPT, D), f32)
    xp = jnp.pad(x, ((0, NP - N), (0, 0)))

    deg = _bincount(src_f, dst_f)
    degt = deg.reshape(2, NW, NP).transpose(2, 0, 1)

    xs, so, si = _tc_scale(degt[:, 0], degt[:, 1], xp)

    p = _agg1(src_f, dst_a, zeros, xs)
    h1s = _tc_l1(p, p, si, so, W1, b1.reshape(1, H), a1.reshape(1, 1))

    q0, q1, q2, q3 = _agg4(src_f, dst_a, zeros,
                           h1s[0], h1s[1], h1s[2], h1s[3])
    Wc = _tc_wc(We2d, Wd)
    ts = _tc_l2(q0, q0, q1, q1, q2, q2, q3, q3, si, so, W2,
                b2.reshape(1, H), a2.reshape(1, 1), Wc)

    r = _agg1(src_f, dst_a, zeros, ts)
    recon = _tc_fin(r, r, si, bd.reshape(1, D))
    return recon[:N]


# speedup vs baseline: 3.2249x; 3.2249x over previous
import jax
import jax.numpy as jnp
from jax import lax
from jax.experimental import pallas as pl
from jax.experimental.pallas import tpu as pltpu
from jax.experimental.pallas import tpu_sc as plsc

N = 10000
E = 320000
D = 128
H = 512

NC = 2
NS = 16
NW = NC * NS
NP = 10240

EPT = 10240
EP = NW * EPT
K = 64
G = EPT // K
RPT = NP // NS
BR = 640
GRID = NP // BR

f32 = jnp.float32


def _mesh():
    return plsc.VectorSubcoreMesh(core_axis_name="c", subcore_axis_name="s")


def _bincount_body(src_hbm, dst_hbm, out_hbm, eidx_s, eidx_d, acc_s, acc_d):
    cid = lax.axis_index("c")
    sid = lax.axis_index("s")
    w = cid * NS + sid
    pltpu.sync_copy(src_hbm.at[pl.ds(w * EPT, EPT)], eidx_s)
    pltpu.sync_copy(dst_hbm.at[pl.ds(w * EPT, EPT)], eidx_d)

    zeros16 = jnp.zeros((16,), f32)

    def zero_blk(i, c):
        acc_s[pl.ds(i * 16, 16)] = zeros16
        acc_d[pl.ds(i * 16, 16)] = zeros16
        return c
    lax.fori_loop(0, NP // 16, zero_blk, 0, unroll=False)

    ones16 = jnp.ones((16,), f32)

    def count_blk(i, c):
        plsc.addupdate_scatter(acc_s, [eidx_s[pl.ds(i * 16, 16)]], ones16)
        plsc.addupdate_scatter(acc_d, [eidx_d[pl.ds(i * 16, 16)]], ones16)
        return c
    lax.fori_loop(0, EPT // 16, count_blk, 0, unroll=False)

    pltpu.sync_copy(acc_s, out_hbm.at[pl.ds(w * NP, NP)])
    pltpu.sync_copy(acc_d, out_hbm.at[pl.ds((NW + w) * NP, NP)])


_bincount = pl.kernel(
    _bincount_body,
    out_type=jax.ShapeDtypeStruct((2 * NW * NP,), f32),
    mesh=_mesh(),
    compiler_params=pltpu.CompilerParams(needs_layout_passes=False),
    scratch_types=[
        pltpu.VMEM((EPT,), jnp.int32),
        pltpu.VMEM((EPT,), jnp.int32),
        pltpu.VMEM((NP,), f32),
        pltpu.VMEM((NP,), f32),
    ],
)


def _make_agg(T):
    def body(src_hbm, dst_hbm, zeros_hbm, *rest):
        tables = rest[:T]
        outs = rest[T:2 * T]
        idx_s, idx_d, rows, sem0, sem1, acc = rest[2 * T:]
        cid = lax.axis_index("c")
        sid = lax.axis_index("s")
        w = cid * NS + sid
        pltpu.sync_copy(src_hbm.at[pl.ds(w * EPT, EPT)], idx_s)
        pltpu.sync_copy(dst_hbm.at[pl.ds(w * G, G)], idx_d)

        def src_at(g):
            return idx_s.at[pl.ds(g * K, K)]

        for t in range(T):
            pltpu.sync_copy(zeros_hbm, acc.at[pl.ds(sid * RPT, RPT)])
            plsc.subcore_barrier()
            tbl = tables[t]
            pltpu.async_copy(tbl.at[src_at(0)], rows.at[0], sem0)

            def pair(h, c):
                g = 2 * h
                pltpu.async_copy(tbl.at[src_at(g + 1)], rows.at[1], sem1)
                pltpu.make_async_copy(tbl.at[src_at(g)], rows.at[0],
                                      sem0).wait()
                pltpu.sync_copy(rows.at[0], acc.at[idx_d.at[g]], add=True)
                gn = jnp.minimum(g + 2, G - 2)
                pltpu.async_copy(tbl.at[src_at(gn)], rows.at[0], sem0)
                pltpu.make_async_copy(tbl.at[src_at(g + 1)], rows.at[1],
                                      sem1).wait()
                pltpu.sync_copy(rows.at[1], acc.at[idx_d.at[g + 1]], add=True)
                return c
            lax.fori_loop(0, G // 2, pair, 0, unroll=False)
            pltpu.make_async_copy(tbl.at[src_at(0)], rows.at[0],
                                  sem0).wait()
            plsc.subcore_barrier()
            pltpu.sync_copy(acc.at[pl.ds(sid * RPT, RPT)],
                            outs[t].at[pl.ds(cid * NP + sid * RPT, RPT)])
    out_type = [jax.ShapeDtypeStruct((2 * NP, D), f32)] * T
    if T == 1:
        out_type = out_type[0]
    return pl.kernel(
        body,
        out_type=out_type,
        mesh=_mesh(),
        compiler_params=pltpu.CompilerParams(needs_layout_passes=False),
        scratch_types=[
            pltpu.VMEM((EPT,), jnp.int32),
            pltpu.VMEM((G, K), jnp.int32),
            pltpu.VMEM((2, K, D), f32),
            pltpu.SemaphoreType.DMA,
            pltpu.SemaphoreType.DMA,
            pltpu.VMEM_SHARED((NP, D), f32),
        ],
    )


_agg1 = _make_agg(1)
_agg4 = _make_agg(4)


def _tc_scale_body(dego, degi, x, xs, so, si):
    so_v = lax.rsqrt(jnp.maximum(
        jnp.sum(dego[...], axis=1, keepdims=True), 1.0))
    si_v = lax.rsqrt(jnp.maximum(
        jnp.sum(degi[...], axis=1, keepdims=True), 1.0))
    xs[...] = x[...] * so_v
    so[...] = so_v
    si[...] = si_v


def _row_spec(width):
    return pl.BlockSpec((BR, width), lambda i: (i, 0))


def _row_spec_off(width, off):
    return pl.BlockSpec((BR, width), lambda i, o=off: (i + o, 0))


def _full_spec(r, c):
    return pl.BlockSpec((r, c), lambda i: (0, 0))


_tc_scale = pl.pallas_call(
    _tc_scale_body,
    grid=(GRID,),
    in_specs=[_row_spec(NW), _row_spec(NW), _row_spec(D)],
    out_specs=[_row_spec(D), _row_spec(1), _row_spec(1)],
    out_shape=[jax.ShapeDtypeStruct((NP, D), f32),
               jax.ShapeDtypeStruct((NP, 1), f32),
               jax.ShapeDtypeStruct((NP, 1), f32)],
)


def _tc_l1_body(p0, p1, si, so, W1, b1, a1, out):
    agg = (p0[...] + p1[...]) * si[...]
    z = jnp.dot(agg, W1[...], preferred_element_type=f32) + b1[...]
    a = a1[0, 0]
    h = jnp.where(z > 0, z, a * z)
    hs = h * so[...]
    for c in range(4):
        out[c] = hs[:, c * 128:(c + 1) * 128]


_tc_l1 = pl.pallas_call(
    _tc_l1_body,
    grid=(GRID,),
    in_specs=[_row_spec(D), _row_spec_off(D, GRID), _row_spec(1), _row_spec(1),
              _full_spec(D, H), _full_spec(1, H), _full_spec(1, 1)],
    out_specs=pl.BlockSpec((4, BR, 128), lambda i: (0, i, 0)),
    out_shape=jax.ShapeDtypeStruct((4, NP, 128), f32),
)


def _tc_wc_body(We2d, Wd, Wc):
    Wc[...] = jnp.dot(We2d[...], Wd[...], preferred_element_type=f32)


_tc_wc = pl.pallas_call(
    _tc_wc_body,
    out_shape=jax.ShapeDtypeStruct((H, D), f32),
)


def _tc_l2_body(q00, q01, q10, q11, q20, q21, q30, q31, si, so, W2, b2, a2,
                Wc, out):
    siv = si[...]
    W2v = W2[...]
    qs = ((q00, q01), (q10, q11), (q20, q21), (q30, q31))
    z = jnp.zeros((BR, H), f32) + b2[...]
    for c in range(4):
        agg_c = (qs[c][0][...] + qs[c][1][...]) * siv
        z = z + jnp.dot(agg_c, W2v[c * 128:(c + 1) * 128, :],
                        preferred_element_type=f32)
    a = a2[0, 0]
    h = jnp.where(z > 0, z, a * z)
    t = jnp.dot(h, Wc[...], preferred_element_type=f32)
    out[...] = t * so[...]


_tc_l2 = pl.pallas_call(
    _tc_l2_body,
    grid=(GRID,),
    in_specs=[_row_spec(D), _row_spec_off(D, GRID)] * 4
    + [_row_spec(1), _row_spec(1), _full_spec(H, H), _full_spec(1, H),
       _full_spec(1, 1), _full_spec(H, D)],
    out_specs=_row_spec(D),
    out_shape=jax.ShapeDtypeStruct((NP, D), f32),
)


def _tc_fin_body(r0, r1, si, bd, out):
    out[...] = (r0[...] + r1[...]) * si[...] + bd[...]


_tc_fin = pl.pallas_call(
    _tc_fin_body,
    grid=(GRID,),
    in_specs=[_row_spec(D), _row_spec_off(D, GRID), _row_spec(1),
              _full_spec(1, D)],
    out_specs=_row_spec(D),
    out_shape=jax.ShapeDtypeStruct((NP, D), f32),
)


def kernel(x, edge_index, W1, b1, a1, W2, b2, a2, We2d, Wd, bd):
    src = edge_index[0]
    dst = edge_index[1]
    pad = N + (jnp.arange(EP - E, dtype=jnp.int32) % (NP - N))
    src_f = jnp.concatenate([src, pad])
    dst_f = jnp.concatenate([dst, pad])
    dst_a = dst_f.reshape(NW * G, K)
    zeros = jnp.zeros((RPT, D), f32)
    xp = jnp.pad(x, ((0, NP - N), (0, 0)))

    deg = _bincount(src_f, dst_f)
    degt = deg.reshape(2, NW, NP).transpose(2, 0, 1)

    xs, so, si = _tc_scale(degt[:, 0], degt[:, 1], xp)

    p = _agg1(src_f, dst_a, zeros, xs)
    h1s = _tc_l1(p, p, si, so, W1, b1.reshape(1, H), a1.reshape(1, 1))

    q0, q1, q2, q3 = _agg4(src_f, dst_a, zeros,
                           h1s[0], h1s[1], h1s[2], h1s[3])
    Wc = _tc_wc(We2d, Wd)
    ts = _tc_l2(q0, q0, q1, q1, q2, q2, q3, q3, si, so, W2,
                b2.reshape(1, H), a2.reshape(1, 1), Wc)

    r = _agg1(src_f, dst_a, zeros, ts)
    recon = _tc_fin(r, r, si, bd.reshape(1, D))
    return recon[:N]


# speedup vs baseline: 3.4437x; 1.0678x over previous
import jax
import jax.numpy as jnp
from jax import lax
from jax.experimental import pallas as pl
from jax.experimental.pallas import tpu as pltpu
from jax.experimental.pallas import tpu_sc as plsc

N = 10000
E = 320000
D = 128
H = 512

NC = 2
NS = 16
NW = NC * NS
NP = 10240

EPT = 10240
EP = NW * EPT
K = 80
G = EPT // K
RPT = NP // NS
BR = 640
GRID = NP // BR

f32 = jnp.float32


def _mesh():
    return plsc.VectorSubcoreMesh(core_axis_name="c", subcore_axis_name="s")


def _bincount_body(src_hbm, dst_hbm, out_hbm, eidx_s, eidx_d, acc_s, acc_d):
    cid = lax.axis_index("c")
    sid = lax.axis_index("s")
    w = cid * NS + sid
    pltpu.sync_copy(src_hbm.at[pl.ds(w * EPT, EPT)], eidx_s)
    pltpu.sync_copy(dst_hbm.at[pl.ds(w * EPT, EPT)], eidx_d)

    zeros16 = jnp.zeros((16,), f32)

    def zero_blk(i, c):
        acc_s[pl.ds(i * 16, 16)] = zeros16
        acc_d[pl.ds(i * 16, 16)] = zeros16
        return c
    lax.fori_loop(0, NP // 16, zero_blk, 0, unroll=False)

    ones16 = jnp.ones((16,), f32)

    def count_blk(i, c):
        plsc.addupdate_scatter(acc_s, [eidx_s[pl.ds(i * 16, 16)]], ones16)
        plsc.addupdate_scatter(acc_d, [eidx_d[pl.ds(i * 16, 16)]], ones16)
        return c
    lax.fori_loop(0, EPT // 16, count_blk, 0, unroll=False)

    pltpu.sync_copy(acc_s, out_hbm.at[pl.ds(w * NP, NP)])
    pltpu.sync_copy(acc_d, out_hbm.at[pl.ds((NW + w) * NP, NP)])


_bincount = pl.kernel(
    _bincount_body,
    out_type=jax.ShapeDtypeStruct((2 * NW * NP,), f32),
    mesh=_mesh(),
    compiler_params=pltpu.CompilerParams(needs_layout_passes=False),
    scratch_types=[
        pltpu.VMEM((EPT,), jnp.int32),
        pltpu.VMEM((EPT,), jnp.int32),
        pltpu.VMEM((NP,), f32),
        pltpu.VMEM((NP,), f32),
    ],
)


def _make_agg(T):
    def body(src_hbm, dst_hbm, zeros_hbm, *rest):
        tables = rest[:T]
        outs = rest[T:2 * T]
        idx_s, idx_d, rows, sem0, sem1, acc = rest[2 * T:]
        cid = lax.axis_index("c")
        sid = lax.axis_index("s")
        w = cid * NS + sid
        pltpu.sync_copy(src_hbm.at[pl.ds(w * EPT, EPT)], idx_s)
        pltpu.sync_copy(dst_hbm.at[pl.ds(w * G, G)], idx_d)

        def src_at(g):
            return idx_s.at[pl.ds(g * K, K)]

        for t in range(T):
            pltpu.sync_copy(zeros_hbm, acc.at[pl.ds(sid * RPT, RPT)])
            plsc.subcore_barrier()
            tbl = tables[t]
            pltpu.async_copy(tbl.at[src_at(0)], rows.at[0], sem0)

            def pair(h, c):
                g = 2 * h
                pltpu.async_copy(tbl.at[src_at(g + 1)], rows.at[1], sem1)
                pltpu.make_async_copy(tbl.at[src_at(g)], rows.at[0],
                                      sem0).wait()
                pltpu.sync_copy(rows.at[0], acc.at[idx_d.at[g]], add=True)
                gn = jnp.minimum(g + 2, G - 2)
                pltpu.async_copy(tbl.at[src_at(gn)], rows.at[0], sem0)
                pltpu.make_async_copy(tbl.at[src_at(g + 1)], rows.at[1],
                                      sem1).wait()
                pltpu.sync_copy(rows.at[1], acc.at[idx_d.at[g + 1]], add=True)
                return c
            lax.fori_loop(0, G // 2, pair, 0, unroll=False)
            pltpu.make_async_copy(tbl.at[src_at(0)], rows.at[0],
                                  sem0).wait()
            plsc.subcore_barrier()
            pltpu.sync_copy(acc.at[pl.ds(sid * RPT, RPT)],
                            outs[t].at[pl.ds(cid * NP + sid * RPT, RPT)])
    out_type = [jax.ShapeDtypeStruct((2 * NP, D), f32)] * T
    if T == 1:
        out_type = out_type[0]
    return pl.kernel(
        body,
        out_type=out_type,
        mesh=_mesh(),
        compiler_params=pltpu.CompilerParams(needs_layout_passes=False),
        scratch_types=[
            pltpu.VMEM((EPT,), jnp.int32),
            pltpu.VMEM((G, K), jnp.int32),
            pltpu.VMEM((2, K, D), f32),
            pltpu.SemaphoreType.DMA,
            pltpu.SemaphoreType.DMA,
            pltpu.VMEM_SHARED((NP, D), f32),
        ],
    )


_agg1 = _make_agg(1)
_agg4 = _make_agg(4)


def _tc_scale_body(dego, degi, x, xs, so, si):
    so_v = lax.rsqrt(jnp.maximum(
        jnp.sum(dego[...], axis=1, keepdims=True), 1.0))
    si_v = lax.rsqrt(jnp.maximum(
        jnp.sum(degi[...], axis=1, keepdims=True), 1.0))
    xs[...] = x[...] * so_v
    so[...] = so_v
    si[...] = si_v


def _row_spec(width):
    return pl.BlockSpec((BR, width), lambda i: (i, 0))


def _row_spec_off(width, off):
    return pl.BlockSpec((BR, width), lambda i, o=off: (i + o, 0))


def _full_spec(r, c):
    return pl.BlockSpec((r, c), lambda i: (0, 0))


_tc_scale = pl.pallas_call(
    _tc_scale_body,
    grid=(GRID,),
    in_specs=[_row_spec(NW), _row_spec(NW), _row_spec(D)],
    out_specs=[_row_spec(D), _row_spec(1), _row_spec(1)],
    out_shape=[jax.ShapeDtypeStruct((NP, D), f32),
               jax.ShapeDtypeStruct((NP, 1), f32),
               jax.ShapeDtypeStruct((NP, 1), f32)],
)


def _tc_l1_body(p0, p1, si, so, W1, b1, a1, out):
    agg = (p0[...] + p1[...]) * si[...]
    z = jnp.dot(agg, W1[...], preferred_element_type=f32) + b1[...]
    a = a1[0, 0]
    h = jnp.where(z > 0, z, a * z)
    hs = h * so[...]
    for c in range(4):
        out[c] = hs[:, c * 128:(c + 1) * 128]


_tc_l1 = pl.pallas_call(
    _tc_l1_body,
    grid=(GRID,),
    in_specs=[_row_spec(D), _row_spec_off(D, GRID), _row_spec(1), _row_spec(1),
              _full_spec(D, H), _full_spec(1, H), _full_spec(1, 1)],
    out_specs=pl.BlockSpec((4, BR, 128), lambda i: (0, i, 0)),
    out_shape=jax.ShapeDtypeStruct((4, NP, 128), f32),
)


def _tc_wc_body(We2d, Wd, Wc):
    Wc[...] = jnp.dot(We2d[...], Wd[...], preferred_element_type=f32)


_tc_wc = pl.pallas_call(
    _tc_wc_body,
    out_shape=jax.ShapeDtypeStruct((H, D), f32),
)


def _tc_l2_body(q00, q01, q10, q11, q20, q21, q30, q31, si, so, W2, b2, a2,
                Wc, out):
    siv = si[...]
    W2v = W2[...]
    qs = ((q00, q01), (q10, q11), (q20, q21), (q30, q31))
    z = jnp.zeros((BR, H), f32) + b2[...]
    for c in range(4):
        agg_c = (qs[c][0][...] + qs[c][1][...]) * siv
        z = z + jnp.dot(agg_c, W2v[c * 128:(c + 1) * 128, :],
                        preferred_element_type=f32)
    a = a2[0, 0]
    h = jnp.where(z > 0, z, a * z)
    t = jnp.dot(h, Wc[...], preferred_element_type=f32)
    out[...] = t * so[...]


_tc_l2 = pl.pallas_call(
    _tc_l2_body,
    grid=(GRID,),
    in_specs=[_row_spec(D), _row_spec_off(D, GRID)] * 4
    + [_row_spec(1), _row_spec(1), _full_spec(H, H), _full_spec(1, H),
       _full_spec(1, 1), _full_spec(H, D)],
    out_specs=_row_spec(D),
    out_shape=jax.ShapeDtypeStruct((NP, D), f32),
)


def _tc_fin_body(r0, r1, si, bd, out):
    out[...] = (r0[...] + r1[...]) * si[...] + bd[...]


_tc_fin = pl.pallas_call(
    _tc_fin_body,
    grid=(GRID,),
    in_specs=[_row_spec(D), _row_spec_off(D, GRID), _row_spec(1),
              _full_spec(1, D)],
    out_specs=_row_spec(D),
    out_shape=jax.ShapeDtypeStruct((NP, D), f32),
)


def kernel(x, edge_index, W1, b1, a1, W2, b2, a2, We2d, Wd, bd):
    src = edge_index[0]
    dst = edge_index[1]
    pad = N + (jnp.arange(EP - E, dtype=jnp.int32) % (NP - N))
    src_f = jnp.concatenate([src, pad])
    dst_f = jnp.concatenate([dst, pad])
    dst_a = dst_f.reshape(NW * G, K)
    zeros = jnp.zeros((RPT, D), f32)
    xp = jnp.pad(x, ((0, NP - N), (0, 0)))

    deg = _bincount(src_f, dst_f)
    degt = deg.reshape(2, NW, NP).transpose(2, 0, 1)

    xs, so, si = _tc_scale(degt[:, 0], degt[:, 1], xp)

    p = _agg1(src_f, dst_a, zeros, xs)
    h1s = _tc_l1(p, p, si, so, W1, b1.reshape(1, H), a1.reshape(1, 1))

    q0, q1, q2, q3 = _agg4(src_f, dst_a, zeros,
                           h1s[0], h1s[1], h1s[2], h1s[3])
    Wc = _tc_wc(We2d, Wd)
    ts = _tc_l2(q0, q0, q1, q1, q2, q2, q3, q3, si, so, W2,
                b2.reshape(1, H), a2.reshape(1, 1), Wc)

    r = _agg1(src_f, dst_a, zeros, ts)
    recon = _tc_fin(r, r, si, bd.reshape(1, D))
    return recon[:N]


# speedup vs baseline: 3.4442x; 1.0001x over previous
import jax
import jax.numpy as jnp
from jax import lax
from jax.experimental import pallas as pl
from jax.experimental.pallas import tpu as pltpu
from jax.experimental.pallas import tpu_sc as plsc

N = 10000
E = 320000
D = 128
H = 512

NC = 2
NS = 16
NW = NC * NS
NP = 10240

EPT = 10240
EP = NW * EPT
K = 80
G = EPT // K
RPT = NP // NS
BR = 640
GRID = NP // BR

f32 = jnp.float32


def _mesh():
    return plsc.VectorSubcoreMesh(core_axis_name="c", subcore_axis_name="s")


def _bincount_body(src_hbm, dst_hbm, out_hbm, eidx_s, eidx_d, acc_s, acc_d):
    cid = lax.axis_index("c")
    sid = lax.axis_index("s")
    w = cid * NS + sid
    pltpu.sync_copy(src_hbm.at[pl.ds(w * EPT, EPT)], eidx_s)
    pltpu.sync_copy(dst_hbm.at[pl.ds(w * EPT, EPT)], eidx_d)

    zeros16 = jnp.zeros((16,), f32)

    def zero_blk(i, c):
        acc_s[pl.ds(i * 16, 16)] = zeros16
        acc_d[pl.ds(i * 16, 16)] = zeros16
        return c
    lax.fori_loop(0, NP // 16, zero_blk, 0, unroll=False)

    ones16 = jnp.ones((16,), f32)

    def count_blk(i, c):
        plsc.addupdate_scatter(acc_s, [eidx_s[pl.ds(i * 16, 16)]], ones16)
        plsc.addupdate_scatter(acc_d, [eidx_d[pl.ds(i * 16, 16)]], ones16)
        return c
    lax.fori_loop(0, EPT // 16, count_blk, 0, unroll=False)

    pltpu.sync_copy(acc_s, out_hbm.at[pl.ds(w * NP, NP)])
    pltpu.sync_copy(acc_d, out_hbm.at[pl.ds((NW + w) * NP, NP)])


_bincount = pl.kernel(
    _bincount_body,
    out_type=jax.ShapeDtypeStruct((2 * NW * NP,), f32),
    mesh=_mesh(),
    compiler_params=pltpu.CompilerParams(needs_layout_passes=False),
    scratch_types=[
        pltpu.VMEM((EPT,), jnp.int32),
        pltpu.VMEM((EPT,), jnp.int32),
        pltpu.VMEM((NP,), f32),
        pltpu.VMEM((NP,), f32),
    ],
)


def _make_agg(T):
    def body(src_hbm, dst_hbm, zeros_hbm, *rest):
        tables = rest[:T]
        outs = rest[T:2 * T]
        idx_s, idx_d, rows, sem0, sem1, acc = rest[2 * T:]
        cid = lax.axis_index("c")
        sid = lax.axis_index("s")
        w = cid * NS + sid
        pltpu.sync_copy(src_hbm.at[pl.ds(w * EPT, EPT)], idx_s)
        pltpu.sync_copy(dst_hbm.at[pl.ds(w * G, G)], idx_d)

        def src_at(g):
            return idx_s.at[pl.ds(g * K, K)]

        for t in range(T):
            pltpu.sync_copy(zeros_hbm, acc.at[pl.ds(sid * RPT, RPT)])
            plsc.subcore_barrier()
            tbl = tables[t]
            pltpu.async_copy(tbl.at[src_at(0)], rows.at[0], sem0)

            def pair(h, c):
                g = 2 * h
                pltpu.async_copy(tbl.at[src_at(g + 1)], rows.at[1], sem1)
                pltpu.make_async_copy(tbl.at[src_at(g)], rows.at[0],
                                      sem0).wait()
                pltpu.sync_copy(rows.at[0], acc.at[idx_d.at[g]], add=True)
                gn = jnp.minimum(g + 2, G - 2)
                pltpu.async_copy(tbl.at[src_at(gn)], rows.at[0], sem0)
                pltpu.make_async_copy(tbl.at[src_at(g + 1)], rows.at[1],
                                      sem1).wait()
                pltpu.sync_copy(rows.at[1], acc.at[idx_d.at[g + 1]], add=True)
                return c
            lax.fori_loop(0, G // 2, pair, 0, unroll=False)
            pltpu.make_async_copy(tbl.at[src_at(0)], rows.at[0],
                                  sem0).wait()
            plsc.subcore_barrier()
            pltpu.sync_copy(acc.at[pl.ds(sid * RPT, RPT)],
                            outs[t].at[pl.ds(cid * NP + sid * RPT, RPT)])
    out_type = [jax.ShapeDtypeStruct((2 * NP, D), f32)] * T
    if T == 1:
        out_type = out_type[0]
    return pl.kernel(
        body,
        out_type=out_type,
        mesh=_mesh(),
        compiler_params=pltpu.CompilerParams(needs_layout_passes=False),
        scratch_types=[
            pltpu.VMEM((EPT,), jnp.int32),
            pltpu.VMEM((G, K), jnp.int32),
            pltpu.VMEM((2, K, D), f32),
            pltpu.SemaphoreType.DMA,
            pltpu.SemaphoreType.DMA,
            pltpu.VMEM_SHARED((NP, D), f32),
        ],
    )


_agg1 = _make_agg(1)
_agg4 = _make_agg(4)


def _tc_scale_body(dego, degi, x, xs, so, si):
    so_v = lax.rsqrt(jnp.maximum(
        jnp.sum(dego[...], axis=1, keepdims=True), 1.0))
    si_v = lax.rsqrt(jnp.maximum(
        jnp.sum(degi[...], axis=1, keepdims=True), 1.0))
    xs[...] = x[...] * so_v
    so[...] = so_v
    si[...] = si_v


def _row_spec(width):
    return pl.BlockSpec((BR, width), lambda i: (i, 0))


def _row_spec_off(width, off):
    return pl.BlockSpec((BR, width), lambda i, o=off: (i + o, 0))


def _full_spec(r, c):
    return pl.BlockSpec((r, c), lambda i: (0, 0))


_tc_scale = pl.pallas_call(
    _tc_scale_body,
    grid=(GRID,),
    in_specs=[_row_spec(NW), _row_spec(NW), _row_spec(D)],
    out_specs=[_row_spec(D), _row_spec(1), _row_spec(1)],
    out_shape=[jax.ShapeDtypeStruct((NP, D), f32),
               jax.ShapeDtypeStruct((NP, 1), f32),
               jax.ShapeDtypeStruct((NP, 1), f32)],
)


def _tc_l1_body(p0, p1, si, so, W1, b1, a1, out):
    agg = (p0[...] + p1[...]) * si[...]
    z = jnp.dot(agg, W1[...], preferred_element_type=f32) + b1[...]
    a = a1[0, 0]
    h = jnp.where(z > 0, z, a * z)
    hs = h * so[...]
    for c in range(4):
        out[c] = hs[:, c * 128:(c + 1) * 128]


_tc_l1 = pl.pallas_call(
    _tc_l1_body,
    grid=(GRID,),
    in_specs=[_row_spec(D), _row_spec_off(D, GRID), _row_spec(1), _row_spec(1),
              _full_spec(D, H), _full_spec(1, H), _full_spec(1, 1)],
    out_specs=pl.BlockSpec((4, BR, 128), lambda i: (0, i, 0)),
    out_shape=jax.ShapeDtypeStruct((4, NP, 128), f32),
)


def _tc_l2_body(q00, q01, q10, q11, q20, q21, q30, q31, si, so, W2, b2, a2,
                We2d, Wd, out, Wc):
    @pl.when(pl.program_id(0) == 0)
    def _():
        Wc[...] = jnp.dot(We2d[...], Wd[...], preferred_element_type=f32)
    siv = si[...]
    W2v = W2[...]
    qs = ((q00, q01), (q10, q11), (q20, q21), (q30, q31))
    z = jnp.zeros((BR, H), f32) + b2[...]
    for c in range(4):
        agg_c = (qs[c][0][...] + qs[c][1][...]) * siv
        z = z + jnp.dot(agg_c, W2v[c * 128:(c + 1) * 128, :],
                        preferred_element_type=f32)
    a = a2[0, 0]
    h = jnp.where(z > 0, z, a * z)
    t = jnp.dot(h, Wc[...], preferred_element_type=f32)
    out[...] = t * so[...]


_tc_l2 = pl.pallas_call(
    _tc_l2_body,
    grid=(GRID,),
    in_specs=[_row_spec(D), _row_spec_off(D, GRID)] * 4
    + [_row_spec(1), _row_spec(1), _full_spec(H, H), _full_spec(1, H),
       _full_spec(1, 1), _full_spec(H, H), _full_spec(H, D)],
    out_specs=_row_spec(D),
    out_shape=jax.ShapeDtypeStruct((NP, D), f32),
    scratch_shapes=[pltpu.VMEM((H, D), f32)],
)


def _tc_fin_body(r0, r1, si, bd, out):
    out[...] = (r0[...] + r1[...]) * si[...] + bd[...]


_tc_fin = pl.pallas_call(
    _tc_fin_body,
    grid=(GRID,),
    in_specs=[_row_spec(D), _row_spec_off(D, GRID), _row_spec(1),
              _full_spec(1, D)],
    out_specs=_row_spec(D),
    out_shape=jax.ShapeDtypeStruct((NP, D), f32),
)


def kernel(x, edge_index, W1, b1, a1, W2, b2, a2, We2d, Wd, bd):
    src = edge_index[0]
    dst = edge_index[1]
    pad = N + (jnp.arange(EP - E, dtype=jnp.int32) % (NP - N))
    src_f = jnp.concatenate([src, pad])
    dst_f = jnp.concatenate([dst, pad])
    dst_a = dst_f.reshape(NW * G, K)
    zeros = jnp.zeros((RPT, D), f32)
    xp = jnp.pad(x, ((0, NP - N), (0, 0)))

    deg = _bincount(src_f, dst_f)
    degt = deg.reshape(2, NW, NP).transpose(2, 0, 1)

    xs, so, si = _tc_scale(degt[:, 0], degt[:, 1], xp)

    p = _agg1(src_f, dst_a, zeros, xs)
    h1s = _tc_l1(p, p, si, so, W1, b1.reshape(1, H), a1.reshape(1, 1))

    q0, q1, q2, q3 = _agg4(src_f, dst_a, zeros,
                           h1s[0], h1s[1], h1s[2], h1s[3])
    ts = _tc_l2(q0, q0, q1, q1, q2, q2, q3, q3, si, so, W2,
                b2.reshape(1, H), a2.reshape(1, 1), We2d, Wd)

    r = _agg1(src_f, dst_a, zeros, ts)
    recon = _tc_fin(r, r, si, bd.reshape(1, D))
    return recon[:N]
